# baseline (speedup 1.0000x reference)
import jax
import jax.numpy as jnp
import numpy as np
from jax import lax
from jax.experimental import pallas as pl
from jax.experimental.pallas import tpu as pltpu
from jax.experimental.pallas import tpu_sc as plsc

NUM_FIELDS_RAW = 39
FIELD_DIM = 2560
EMBED_DIM = 16
BATCH = 4096

F = 30
FPAD = 32
TOTAL = F * FIELD_DIM
ROWLEN = F * EMBED_DIM
NC, NS, L = 2, 16, 16
NW = NC * NS
BPW = BATCH // NW


def _field_offsets_i32():
    sel = np.full(NUM_FIELDS_RAW, FIELD_DIM, dtype=np.int64)
    sel = np.hstack((sel[:3], sel[4:8], sel[10:15], sel[17:19], sel[21:24], sel[26:]))
    return np.array((0, *np.cumsum(sel)[:-1]), dtype=np.int32)


def _select_cols(x):
    return jnp.concatenate(
        (x[:, :3], x[:, 4:8], x[:, 10:15], x[:, 17:19], x[:, 21:24], x[:, 26:]),
        axis=1)


def _ffm_kernel(big_hbm, w_hbm, xi_hbm, bias_hbm, out_hbm,
                xi_v, w_all_v, idx0, idx1, rows0, rows1,
                z_v, bias_v, sem0, sem1):
    idx_bufs = (idx0, idx1)
    rows_bufs = (rows0, rows1)
    sems = (sem0, sem1)
    wid = lax.axis_index("s") * NC + lax.axis_index("c")
    base = wid * BPW
    pltpu.sync_copy(xi_hbm.at[pl.ds(base, BPW)], xi_v)
    pltpu.sync_copy(bias_hbm, bias_v)
    pltpu.sync_copy(w_hbm, w_all_v)

    lanes = lax.iota(jnp.int32, L)
    pa = jnp.where(lanes < 2, lanes + 14, 0)
    pb = jnp.where(lanes < 2, 0, lanes - 2)

    def build_and_fire(s, idx_v, rows_v, sem):
        xa = xi_v[s, pl.ds(0, L)]
        xb = xi_v[s, pl.ds(L, L)]
        idx_v[pl.ds(0, L)] = xa
        tail = jnp.where(
            lanes < 2,
            jnp.take_along_axis(xa, pa, axis=0, mode="promise_in_bounds"),
            jnp.take_along_axis(xb, pb, axis=0, mode="promise_in_bounds"))
        idx_v[pl.ds(F - L, L)] = tail
        return pltpu.async_copy(big_hbm.at[idx_v], rows_v, sem)

    def compute(s, rows_v, zvec):
        xa = xi_v[s, pl.ds(0, L)]
        xb = xi_v[s, pl.ds(L, L)]
        ga = plsc.load_gather(w_all_v, [xa])
        gb = plsc.load_gather(w_all_v, [xb])
        accw = ga + jnp.where(lanes < F - L, gb, 0.0)

        def pair_i(i, accP):
            ai = i * EMBED_DIM

            def pj(j, a):
                return a + (rows_v[j, pl.ds(ai, L)]
                            * rows_v[i, pl.ds(j * EMBED_DIM, L)])

            return lax.fori_loop(0, F, pj, accP, unroll=6)

        accP = lax.fori_loop(0, F, pair_i, jnp.zeros((L,), jnp.float32))

        def diag_i(i, aD):
            d = rows_v[i, pl.ds(i * EMBED_DIM, L)]
            return aD + d * d

        accD = lax.fori_loop(0, F, diag_i, jnp.zeros((L,), jnp.float32),
                             unroll=6)

        acc = accw + 0.5 * (accP - accD)
        for sh in (1, 2, 4, 8):
            acc = acc + jnp.take_along_axis(
                acc, lanes ^ sh, axis=0, mode="promise_in_bounds")
        lane = s % L
        zvec = jnp.where(lanes == lane, acc, zvec)

        @pl.when(lane == L - 1)
        def _():
            z_v[pl.ds(pl.multiple_of((s // L) * L, L), L)] = zvec

        return zvec

    h_first = build_and_fire(0, idx0, rows0, sem0)

    def pair_body(g, zvec):
        s0 = g * 2
        build_and_fire(s0 + 1, idx1, rows1, sem1)
        pltpu.make_async_copy(big_hbm.at[idx0], rows0, sem0).wait()
        zvec = compute(s0, rows0, zvec)

        @pl.when(s0 + 2 < BPW)
        def _():
            build_and_fire(s0 + 2, idx0, rows0, sem0)

        pltpu.make_async_copy(big_hbm.at[idx1], rows1, sem1).wait()
        zvec = compute(s0 + 1, rows1, zvec)
        return zvec

    del h_first
    lax.fori_loop(0, BPW // 2, pair_body, jnp.zeros((L,), jnp.float32))

    for g in range(BPW // L):
        zz = z_v[pl.ds(g * L, L)]
        z_v[pl.ds(g * L, L)] = 1.0 / (1.0 + jnp.exp(-(zz + bias_v[...])))
    pltpu.sync_copy(z_v, out_hbm.at[pl.ds(base, BPW)])


@jax.jit
def _run(big, w1d, xi_pad, bias16):
    mesh = plsc.VectorSubcoreMesh(
        core_axis_name="c", subcore_axis_name="s", num_cores=NC, num_subcores=NS)
    return pl.kernel(
        _ffm_kernel,
        out_type=jax.ShapeDtypeStruct((BATCH,), jnp.float32),
        mesh=mesh,
        compiler_params=pltpu.CompilerParams(
            use_tc_tiling_on_sc=False, needs_layout_passes=False),
        scratch_types=[
            pltpu.VMEM((BPW, FPAD), jnp.int32),
            pltpu.VMEM((TOTAL,), jnp.float32),
            pltpu.VMEM((F,), jnp.int32),
            pltpu.VMEM((F,), jnp.int32),
            pltpu.VMEM((F, ROWLEN), jnp.float32),
            pltpu.VMEM((F, ROWLEN), jnp.float32),
            pltpu.VMEM((BPW,), jnp.float32),
            pltpu.VMEM((L,), jnp.float32),
            pltpu.SemaphoreType.DMA,
            pltpu.SemaphoreType.DMA,
        ],
    )(big, w1d, xi_pad, bias16)


def kernel(x, additional, W_lin, bias, ffm_tables):
    offsets = jnp.asarray(_field_offsets_i32())
    xi = _select_cols(x).astype(jnp.int32) + offsets[None, :]
    xi_pad = jnp.pad(xi, ((0, 0), (0, FPAD - F)))
    big = jnp.swapaxes(ffm_tables, 0, 1).reshape(TOTAL, ROWLEN)
    bias16 = jnp.broadcast_to(bias.astype(jnp.float32), (L,))
    return _run(big, W_lin.astype(jnp.float32).reshape(TOTAL), xi_pad, bias16)
